# baseline (speedup 1.0000x reference)
import functools

import jax
import jax.numpy as jnp
from jax.experimental import pallas as pl

_EPS = 1e-12


def _fused_body(x_ref, qf_ref, glf_ref, qs_ref, ls_ref, out_ref):
    j = pl.program_id(0)
    n = qs_ref.shape[0]
    bj = qs_ref.shape[-1]
    qs = qs_ref[...].reshape(n, bj)
    t = jnp.dot(x_ref[...], qs, preferred_element_type=jnp.float32)
    x_hat = jax.lax.dot_general(
        qf_ref[...], t, (((0,), (0,)), ((), ())),
        preferred_element_type=jnp.float32)
    ls = ls_ref[...].reshape(1, bj)
    r = 1.0 / (1.0 - glf_ref[...] * ls)
    y = jnp.dot(qf_ref[...], r * x_hat, preferred_element_type=jnp.float32)
    z = jax.lax.dot_general(
        y, qs, (((1,), (1,)), ((), ())),
        preferred_element_type=jnp.float32)

    @pl.when(j == 0)
    def _init():
        out_ref[...] = jnp.zeros_like(out_ref)

    out_ref[...] += z


@functools.partial(jax.jit, static_argnames=())
def kernel(X, F, Q_S, Lambda_S, gamma):
    m, n = X.shape
    FF = F.T @ F
    G = FF / (jnp.linalg.norm(FF) + _EPS)
    lam_f, Q_F = jnp.linalg.eigh(G)
    glf = (gamma * lam_f).reshape(m, 1).astype(jnp.float32)

    BJ = 250
    nj = n // BJ
    qs4 = Q_S.reshape(n, nj, 1, BJ)
    ls3 = Lambda_S.astype(jnp.float32).reshape(nj, 1, BJ)

    Z = pl.pallas_call(
        _fused_body,
        grid=(nj,),
        in_specs=[
            pl.BlockSpec((m, n), lambda j: (0, 0)),
            pl.BlockSpec((m, m), lambda j: (0, 0)),
            pl.BlockSpec((m, 1), lambda j: (0, 0)),
            pl.BlockSpec((n, 1, 1, BJ), lambda j: (0, j, 0, 0)),
            pl.BlockSpec((1, 1, BJ), lambda j: (j, 0, 0)),
        ],
        out_specs=pl.BlockSpec((m, n), lambda j: (0, 0)),
        out_shape=jax.ShapeDtypeStruct((m, n), jnp.float32),
    )(X, Q_F, glf, qs4, ls3)
    return Z


# speedup vs baseline: 9.8569x; 9.8569x over previous
import jax
import jax.numpy as jnp
from jax.experimental import pallas as pl
from jax.experimental.pallas import tpu as pltpu

_EPS = 1e-12
_K = 16
_BJ = 384


def _fused_body(c_ref, x_ref, g_ref, ls_ref, qs_ref, out_ref):
    j = pl.program_id(0)
    n = x_ref.shape[1]
    bj = qs_ref.shape[1]
    col = jax.lax.broadcasted_iota(jnp.int32, (1, bj), 1)
    valid = col < (n - j * bj)
    qs = jnp.where(valid, qs_ref[...], 0.0)
    ls = jnp.where(valid, ls_ref[...], 0.0)
    g = g_ref[...]

    v = jnp.dot(x_ref[...], qs, preferred_element_type=jnp.float32)

    bc = c_ref[_K] * v
    bp = jnp.zeros_like(v)
    for k in range(_K - 1, 0, -1):
        bn = c_ref[k] * v + 2.0 * jnp.dot(
            g, bc, preferred_element_type=jnp.float32) * ls - bp
        bp = bc
        bc = bn
    y = c_ref[0] * v + jnp.dot(
        g, bc, preferred_element_type=jnp.float32) * ls - bp

    z = jax.lax.dot_general(
        y, qs, (((1,), (1,)), ((), ())), preferred_element_type=jnp.float32)

    @pl.when(j == 0)
    def _init():
        out_ref[...] = jnp.zeros_like(out_ref)

    out_ref[...] += z


def kernel(X, F, Q_S, Lambda_S, gamma):
    m, n = X.shape
    FF = F.T @ F
    G = (FF / (jnp.linalg.norm(FF) + _EPS)).astype(jnp.float32)

    gam = jnp.asarray(gamma, jnp.float32)
    a = 1.0 / gam
    s = jnp.sqrt(a * a - 1.0)
    q = a - s
    scale = 2.0 / (gam * s)
    ks = jnp.arange(_K + 1, dtype=jnp.float32)
    c = scale * q ** ks
    c = c.at[0].multiply(0.5)

    ls_row = Lambda_S.astype(jnp.float32).reshape(1, n)
    nj = pl.cdiv(n, _BJ)

    Z = pl.pallas_call(
        _fused_body,
        grid=(nj,),
        in_specs=[
            pl.BlockSpec(memory_space=pltpu.SMEM),
            pl.BlockSpec((m, n), lambda j: (0, 0)),
            pl.BlockSpec((m, m), lambda j: (0, 0)),
            pl.BlockSpec((1, _BJ), lambda j: (0, j)),
            pl.BlockSpec((n, _BJ), lambda j: (0, j)),
        ],
        out_specs=pl.BlockSpec((m, n), lambda j: (0, 0)),
        out_shape=jax.ShapeDtypeStruct((m, n), jnp.float32),
    )(c, X, G, ls_row, Q_S)
    return Z
